# Initial kernel scaffold; baseline (speedup 1.0000x reference)
import functools

import jax
import jax.numpy as jnp
from jax import lax
from jax.experimental import pallas as pl
from jax.experimental.pallas import tpu as pltpu
from jax.experimental.pallas import tpu_sc as plsc

STATE = 8

_NC = 2
_NS = 16
_NW = _NC * _NS

_CHUNK = 128


def _sc_segment_sum(x, src_r, dst_r, zeros, n_pad, k_chunks):
    n = x.shape[0]
    zr = n_pad // _NS
    orr = n // _NS

    @functools.partial(
        pl.kernel,
        out_type=jax.ShapeDtypeStruct((2 * n, STATE), jnp.float32),
        mesh=plsc.VectorSubcoreMesh(core_axis_name="c", subcore_axis_name="s"),
        scratch_types=[
            pltpu.VMEM((k_chunks, _CHUNK), jnp.int32),
            pltpu.VMEM((k_chunks, _CHUNK), jnp.int32),
            pltpu.VMEM((_CHUNK, STATE), jnp.float32),
            pltpu.VMEM_SHARED((n_pad, STATE), jnp.float32),
            pltpu.SemaphoreType.DMA,
        ],
    )
    def sc_kernel(x_hbm, src_hbm, dst_hbm, zeros_hbm, p_hbm,
                  src_v, dst_v, rows_v, agg_sh, sem):
        c = lax.axis_index("c")
        s = lax.axis_index("s")
        wid = s * _NC + c

        pltpu.sync_copy(zeros_hbm.at[pl.ds(s * zr, zr)],
                        agg_sh.at[pl.ds(s * zr, zr)])
        plsc.subcore_barrier()

        pltpu.sync_copy(src_hbm.at[wid], src_v)
        pltpu.sync_copy(dst_hbm.at[wid], dst_v)

        def body(j, carry):
            pltpu.async_copy(x_hbm.at[src_v.at[j]], rows_v, sem).wait()
            pltpu.sync_copy(rows_v, agg_sh.at[dst_v.at[j]], add=True)
            return carry

        lax.fori_loop(0, k_chunks, body, 0)
        plsc.subcore_barrier()

        pltpu.sync_copy(agg_sh.at[pl.ds(s * orr, orr)],
                        p_hbm.at[pl.ds(c * n + s * orr, orr)])

    return sc_kernel(x, src_r, dst_r, zeros)


def _mlp_block(x_ref, p0_ref, p1_ref, w1a_ref, w1b_ref, b1_ref, w2_ref,
               b2_ref, o_ref):
    agg = (p0_ref[...] + p1_ref[...]) * (1.0 / 6.0)
    pre = (jnp.dot(x_ref[...], w1a_ref[...], preferred_element_type=jnp.float32)
           + jnp.dot(agg, w1b_ref[...], preferred_element_type=jnp.float32)
           + b1_ref[...])
    h = jnp.tanh(pre)
    o_ref[...] = jnp.tanh(
        jnp.dot(h, w2_ref[...], preferred_element_type=jnp.float32)
        + b2_ref[...])


def _mlp(x, partials, w1, b1, w2, b2):
    n = x.shape[0]
    hid = w1.shape[1]
    bn = 5000
    assert n % bn == 0
    grid = (n // bn,)
    w1a = w1[:STATE]
    w1b = w1[STATE:]
    b1r = b1.reshape(1, hid)
    b2r = b2.reshape(1, STATE)
    p0 = partials[:n]
    p1 = partials[n:]

    const = lambda shape: pl.BlockSpec(shape, lambda i: (0, 0))
    rows = lambda width: pl.BlockSpec((bn, width), lambda i: (i, 0))
    return pl.pallas_call(
        _mlp_block,
        grid=grid,
        in_specs=[
            rows(STATE), rows(STATE), rows(STATE),
            const((STATE, hid)), const((STATE, hid)), const((1, hid)),
            const((hid, STATE)), const((1, STATE)),
        ],
        out_specs=rows(STATE),
        out_shape=jax.ShapeDtypeStruct((n, STATE), jnp.float32),
    )(x, p0, p1, w1a, w1b, b1r, w2, b2r)


def kernel(x, edge_index, W1, b1, W2, b2):
    n = x.shape[0]
    e = edge_index.shape[1]

    per_w = -(-e // _NW)
    k_chunks = -(-per_w // _CHUNK)
    e_pad = _NW * k_chunks * _CHUNK
    pad = e_pad - e

    src = edge_index[0]
    dst = edge_index[1]
    if pad:
        src = jnp.concatenate([src, jnp.zeros((pad,), jnp.int32)])
        dst = jnp.concatenate([dst, jnp.full((pad,), n, jnp.int32)])
    src_r = src.reshape(_NW, k_chunks, _CHUNK)
    dst_r = dst.reshape(_NW, k_chunks, _CHUNK)

    n_pad = -(-(n + (1 if pad else 0)) // _NS) * _NS
    zeros = jnp.zeros((n_pad, STATE), jnp.float32)

    partials = _sc_segment_sum(x, src_r, dst_r, zeros, n_pad, k_chunks)
    return _mlp(x, partials, W1, b1, W2, b2)


# speedup vs baseline: 10.0160x; 10.0160x over previous
import functools

import jax
import jax.numpy as jnp
from jax import lax
from jax.experimental import pallas as pl
from jax.experimental.pallas import tpu as pltpu
from jax.experimental.pallas import tpu_sc as plsc

STATE = 8

_NC = 2
_NS = 16
_NW = _NC * _NS

_CHUNK = 128


def _sc_segment_sum(x, src_r, dst_r, zeros, n_pad, k_chunks):
    zr = n_pad // _NS

    @functools.partial(
        pl.kernel,
        out_type=jax.ShapeDtypeStruct((2 * n_pad, STATE), jnp.float32),
        mesh=plsc.VectorSubcoreMesh(core_axis_name="c", subcore_axis_name="s"),
        compiler_params=pltpu.CompilerParams(use_tc_tiling_on_sc=False),
        scratch_types=[
            pltpu.VMEM((k_chunks, _CHUNK), jnp.int32),
            pltpu.VMEM((k_chunks, _CHUNK), jnp.int32),
            pltpu.VMEM((_CHUNK, STATE), jnp.float32),
            pltpu.VMEM_SHARED((n_pad, STATE), jnp.float32),
            pltpu.SemaphoreType.DMA,
        ],
    )
    def sc_kernel(x_hbm, src_hbm, dst_hbm, zeros_hbm, p_hbm,
                  src_v, dst_v, rows_v, agg_sh, sem):
        c = lax.axis_index("c")
        s = lax.axis_index("s")
        wid = s * _NC + c

        pltpu.sync_copy(zeros_hbm.at[pl.ds(s * zr, zr)],
                        agg_sh.at[pl.ds(s * zr, zr)])
        plsc.subcore_barrier()

        pltpu.sync_copy(src_hbm.at[wid], src_v)
        pltpu.sync_copy(dst_hbm.at[wid], dst_v)

        def body(j, carry):
            pltpu.async_copy(x_hbm.at[src_v.at[j]], rows_v, sem).wait()
            pltpu.sync_copy(rows_v, agg_sh.at[dst_v.at[j]], add=True)
            return carry

        lax.fori_loop(0, k_chunks, body, 0)
        plsc.subcore_barrier()

        pltpu.sync_copy(agg_sh.at[pl.ds(s * zr, zr)],
                        p_hbm.at[pl.ds(c * n_pad + s * zr, zr)])

    return sc_kernel(x, src_r, dst_r, zeros)


def _mlp_block(x_ref, p0_ref, p1_ref, w1a_ref, w1b_ref, b1_ref, w2_ref,
               b2_ref, o_ref):
    agg = (p0_ref[...] + p1_ref[...]) * (1.0 / 6.0)
    pre = (jnp.dot(x_ref[...], w1a_ref[...], preferred_element_type=jnp.float32)
           + jnp.dot(agg, w1b_ref[...], preferred_element_type=jnp.float32)
           + b1_ref[...])
    h = jnp.tanh(pre)
    o_ref[...] = jnp.tanh(
        jnp.dot(h, w2_ref[...], preferred_element_type=jnp.float32)
        + b2_ref[...])


def _mlp(x, partials, n_pad, w1, b1, w2, b2):
    n = x.shape[0]
    hid = w1.shape[1]
    bn = 5000
    assert n % bn == 0
    grid = (n // bn,)
    w1a = w1[:STATE]
    w1b = w1[STATE:]
    b1r = b1.reshape(1, hid)
    b2r = b2.reshape(1, STATE)
    p0 = partials[:n]
    p1 = partials[n_pad:n_pad + n]

    const = lambda shape: pl.BlockSpec(shape, lambda i: (0, 0))
    rows = lambda width: pl.BlockSpec((bn, width), lambda i: (i, 0))
    return pl.pallas_call(
        _mlp_block,
        grid=grid,
        in_specs=[
            rows(STATE), rows(STATE), rows(STATE),
            const((STATE, hid)), const((STATE, hid)), const((1, hid)),
            const((hid, STATE)), const((1, STATE)),
        ],
        out_specs=rows(STATE),
        out_shape=jax.ShapeDtypeStruct((n, STATE), jnp.float32),
    )(x, p0, p1, w1a, w1b, b1r, w2, b2r)


def kernel(x, edge_index, W1, b1, W2, b2):
    n = x.shape[0]
    e = edge_index.shape[1]

    per_w = -(-e // _NW)
    k_chunks = -(-per_w // _CHUNK)
    e_pad = _NW * k_chunks * _CHUNK
    pad = e_pad - e

    src = edge_index[0]
    dst = edge_index[1]
    if pad:
        src = jnp.concatenate([src, jnp.zeros((pad,), jnp.int32)])
        dst = jnp.concatenate([dst, jnp.full((pad,), n, jnp.int32)])
    src_r = src.reshape(_NW, k_chunks, _CHUNK)
    dst_r = dst.reshape(_NW, k_chunks, _CHUNK)

    n_pad = -(-(n + (1 if pad else 0)) // (_NS * 8)) * (_NS * 8)
    zeros = jnp.zeros((n_pad, STATE), jnp.float32)

    partials = _sc_segment_sum(x, src_r, dst_r, zeros, n_pad, k_chunks)
    return _mlp(x, partials, n_pad, W1, b1, W2, b2)


# speedup vs baseline: 13.3787x; 1.3357x over previous
import functools

import jax
import jax.numpy as jnp
from jax import lax
from jax.experimental import pallas as pl
from jax.experimental.pallas import tpu as pltpu
from jax.experimental.pallas import tpu_sc as plsc

STATE = 8

_NC = 2
_NS = 16
_NW = _NC * _NS

_CHUNK = 125


def _sc_segment_sum(x, src_r, dst_r, zeros, k_chunks):
    n = x.shape[0]
    zr = n // _NS

    @functools.partial(
        pl.kernel,
        out_type=jax.ShapeDtypeStruct((2 * n, STATE), jnp.float32),
        mesh=plsc.VectorSubcoreMesh(core_axis_name="c", subcore_axis_name="s"),
        compiler_params=pltpu.CompilerParams(use_tc_tiling_on_sc=False),
        scratch_types=[
            pltpu.VMEM((k_chunks, _CHUNK), jnp.int32),
            pltpu.VMEM((k_chunks, _CHUNK), jnp.int32),
            pltpu.VMEM((_CHUNK, STATE), jnp.float32),
            pltpu.VMEM_SHARED((n, STATE), jnp.float32),
            pltpu.SemaphoreType.DMA,
        ],
    )
    def sc_kernel(x_hbm, src_hbm, dst_hbm, zeros_hbm, p_hbm,
                  src_v, dst_v, rows_v, agg_sh, sem):
        c = lax.axis_index("c")
        s = lax.axis_index("s")
        wid = s * _NC + c

        pltpu.sync_copy(zeros_hbm.at[pl.ds(s * zr, zr)],
                        agg_sh.at[pl.ds(s * zr, zr)])
        plsc.subcore_barrier()

        pltpu.sync_copy(src_hbm.at[wid], src_v)
        pltpu.sync_copy(dst_hbm.at[wid], dst_v)

        def body(j, carry):
            pltpu.async_copy(x_hbm.at[src_v.at[j]], rows_v, sem).wait()
            pltpu.sync_copy(rows_v, agg_sh.at[dst_v.at[j]], add=True)
            return carry

        lax.fori_loop(0, k_chunks, body, 0)
        plsc.subcore_barrier()

        pltpu.sync_copy(agg_sh.at[pl.ds(s * zr, zr)],
                        p_hbm.at[pl.ds(c * n + s * zr, zr)])

    return sc_kernel(x, src_r, dst_r, zeros)


def _mlp_block(x_ref, p0_ref, p1_ref, w1a_ref, w1b_ref, b1_ref, w2_ref,
               b2_ref, o_ref):
    agg = (p0_ref[0] + p1_ref[0]) * (1.0 / 6.0)
    pre = (jnp.dot(x_ref[...], w1a_ref[...], preferred_element_type=jnp.float32)
           + jnp.dot(agg, w1b_ref[...], preferred_element_type=jnp.float32)
           + b1_ref[...])
    h = jnp.tanh(pre)
    o_ref[...] = jnp.tanh(
        jnp.dot(h, w2_ref[...], preferred_element_type=jnp.float32)
        + b2_ref[...])


def _block_diag(w, copies):
    a, b = w.shape
    eye = jnp.eye(copies, dtype=w.dtype)
    return (eye[:, None, :, None] * w[None, :, None, :]).reshape(
        copies * a, copies * b)


def _mlp_wide(xw, pw, w1, b1, w2, b2):
    nw = xw.shape[0]
    hid = w1.shape[1]
    cells = 128 // STATE
    bw = nw
    grid = (nw // bw,)
    p3 = pw.reshape(2, nw, 128)

    w1a = _block_diag(w1[:STATE], cells)
    w1b = _block_diag(w1[STATE:], cells)
    w2d = _block_diag(w2, cells)
    b1t = jnp.tile(b1, cells).reshape(1, cells * hid)
    b2t = jnp.tile(b2, cells).reshape(1, cells * STATE)

    const = lambda shape: pl.BlockSpec(shape, lambda i: (0, 0))
    return pl.pallas_call(
        _mlp_block,
        grid=grid,
        in_specs=[
            pl.BlockSpec((bw, 128), lambda i: (i, 0)),
            pl.BlockSpec((1, bw, 128), lambda i: (0, i, 0)),
            pl.BlockSpec((1, bw, 128), lambda i: (1, i, 0)),
            const((128, cells * hid)), const((128, cells * hid)),
            const((1, cells * hid)),
            const((cells * hid, 128)), const((1, 128)),
        ],
        out_specs=pl.BlockSpec((bw, 128), lambda i: (i, 0)),
        out_shape=jax.ShapeDtypeStruct((nw, 128), jnp.float32),
    )(xw, p3, p3, w1a, w1b, b1t, w2d, b2t)


def kernel(x, edge_index, W1, b1, W2, b2):
    n = x.shape[0]
    e = edge_index.shape[1]
    assert e % (_NW * _CHUNK) == 0 and n % (128 // STATE) == 0 and n % _NS == 0
    k_chunks = e // (_NW * _CHUNK)

    src_r = edge_index[0].reshape(_NW, k_chunks, _CHUNK)
    dst_r = edge_index[1].reshape(_NW, k_chunks, _CHUNK)
    zeros = jnp.zeros((n, STATE), jnp.float32)

    xw = x.reshape(n * STATE // 128, 128)
    x_lin = xw.reshape(n, STATE)

    partials = _sc_segment_sum(x_lin, src_r, dst_r, zeros, k_chunks)
    pw = partials.reshape(2 * n * STATE // 128, 128)
    out_w = _mlp_wide(xw, pw, W1, b1, W2, b2)
    return out_w.reshape(n, STATE)


# speedup vs baseline: 19.5186x; 1.4589x over previous
import functools

import jax
import jax.numpy as jnp
from jax import lax
from jax.experimental import pallas as pl
from jax.experimental.pallas import tpu as pltpu
from jax.experimental.pallas import tpu_sc as plsc

STATE = 8

_NC = 2
_NS = 16
_NW = _NC * _NS

_CHUNK = 125


_UNROLL = 5


def _sc_segment_sum(x, ei_r, zeros, k_chunks):
    n = x.shape[0]
    zr = n // _NS
    u_n = _UNROLL
    outer = k_chunks // u_n

    @functools.partial(
        pl.kernel,
        out_type=jax.ShapeDtypeStruct((2 * n, STATE), jnp.float32),
        mesh=plsc.VectorSubcoreMesh(core_axis_name="c", subcore_axis_name="s"),
        compiler_params=pltpu.CompilerParams(use_tc_tiling_on_sc=False),
        scratch_types=[
            pltpu.VMEM((k_chunks, _CHUNK), jnp.int32),
            pltpu.VMEM((k_chunks, _CHUNK), jnp.int32),
            pltpu.VMEM((u_n, _CHUNK, STATE), jnp.float32),
            pltpu.VMEM_SHARED((n, STATE), jnp.float32),
            pltpu.SemaphoreType.DMA,
            pltpu.SemaphoreType.DMA,
        ],
    )
    def sc_kernel(x_hbm, ei_hbm, zeros_hbm, p_hbm,
                  src_v, dst_v, rows_v, agg_sh, gsem, ssem):
        c = lax.axis_index("c")
        s = lax.axis_index("s")
        wid = s * _NC + c

        pltpu.sync_copy(zeros_hbm.at[pl.ds(s * zr, zr)],
                        agg_sh.at[pl.ds(s * zr, zr)])
        plsc.subcore_barrier()

        pltpu.sync_copy(ei_hbm.at[0, wid], src_v)
        pltpu.sync_copy(ei_hbm.at[1, wid], dst_v)

        def scatter_descr(u, j):
            return pltpu.make_async_copy(
                rows_v.at[u], agg_sh.at[dst_v.at[j]], ssem)

        def body(jo, carry):
            base = jo * u_n

            @pl.when(jo > 0)
            def _():
                for u in range(u_n):
                    scatter_descr(u, base + u).wait()

            gets = [
                pltpu.async_copy(x_hbm.at[src_v.at[base + u]],
                                 rows_v.at[u], gsem)
                for u in range(u_n)
            ]
            for u in range(u_n):
                gets[u].wait()
                scatter_descr(u, base + u).start(add=True)
            return carry

        lax.fori_loop(0, outer, body, 0)
        for u in range(u_n):
            scatter_descr(u, u).wait()
        plsc.subcore_barrier()

        pltpu.sync_copy(agg_sh.at[pl.ds(s * zr, zr)],
                        p_hbm.at[pl.ds(c * n + s * zr, zr)])

    return sc_kernel(x, ei_r, zeros)


def _mlp_block(x_ref, p0_ref, p1_ref, w1a_ref, w1b_ref, b1_ref, w2_ref,
               b2_ref, o_ref):
    agg = (p0_ref[0] + p1_ref[0]) * (1.0 / 6.0)
    pre = (jnp.dot(x_ref[...], w1a_ref[...], preferred_element_type=jnp.float32)
           + jnp.dot(agg, w1b_ref[...], preferred_element_type=jnp.float32)
           + b1_ref[...])
    h = jnp.tanh(pre)
    o_ref[...] = jnp.tanh(
        jnp.dot(h, w2_ref[...], preferred_element_type=jnp.float32)
        + b2_ref[...])


def _block_diag(w, copies):
    a, b = w.shape
    eye = jnp.eye(copies, dtype=w.dtype)
    return (eye[:, None, :, None] * w[None, :, None, :]).reshape(
        copies * a, copies * b)


def _mlp_wide(xw, pw, w1, b1, w2, b2):
    nw = xw.shape[0]
    hid = w1.shape[1]
    cells = 128 // STATE
    bw = nw
    grid = (nw // bw,)
    p3 = pw.reshape(2, nw, 128)

    w1a = _block_diag(w1[:STATE], cells)
    w1b = _block_diag(w1[STATE:], cells)
    w2d = _block_diag(w2, cells)
    b1t = jnp.tile(b1, cells).reshape(1, cells * hid)
    b2t = jnp.tile(b2, cells).reshape(1, cells * STATE)

    const = lambda shape: pl.BlockSpec(shape, lambda i: (0, 0))
    return pl.pallas_call(
        _mlp_block,
        grid=grid,
        in_specs=[
            pl.BlockSpec((bw, 128), lambda i: (i, 0)),
            pl.BlockSpec((1, bw, 128), lambda i: (0, i, 0)),
            pl.BlockSpec((1, bw, 128), lambda i: (1, i, 0)),
            const((128, cells * hid)), const((128, cells * hid)),
            const((1, cells * hid)),
            const((cells * hid, 128)), const((1, 128)),
        ],
        out_specs=pl.BlockSpec((bw, 128), lambda i: (i, 0)),
        out_shape=jax.ShapeDtypeStruct((nw, 128), jnp.float32),
    )(xw, p3, p3, w1a, w1b, b1t, w2d, b2t)


def kernel(x, edge_index, W1, b1, W2, b2):
    n = x.shape[0]
    e = edge_index.shape[1]
    assert e % (_NW * _CHUNK * _UNROLL) == 0
    assert n % (128 // STATE) == 0 and n % _NS == 0
    k_chunks = e // (_NW * _CHUNK)

    ei_r = edge_index.reshape(2, _NW, k_chunks, _CHUNK)
    zeros = jnp.zeros((n, STATE), jnp.float32)

    xw = x.reshape(n * STATE // 128, 128)
    x_lin = xw.reshape(n, STATE)

    partials = _sc_segment_sum(x_lin, ei_r, zeros, k_chunks)
    pw = partials.reshape(2 * n * STATE // 128, 128)
    out_w = _mlp_wide(xw, pw, W1, b1, W2, b2)
    return out_w.reshape(n, STATE)


# speedup vs baseline: 25.2453x; 1.2934x over previous
import functools

import jax
import jax.numpy as jnp
from jax import lax
from jax.experimental import pallas as pl
from jax.experimental.pallas import tpu as pltpu
from jax.experimental.pallas import tpu_sc as plsc

STATE = 8

_NC = 2
_NS = 16
_NW = _NC * _NS

_CHUNK = 125
_UNROLL = 5
_SB = 3136


def _sc_transpose(xt, n):
    per_w = n // _NW

    @functools.partial(
        pl.kernel,
        out_type=jax.ShapeDtypeStruct((n, STATE), jnp.float32),
        mesh=plsc.VectorSubcoreMesh(core_axis_name="c", subcore_axis_name="s"),
        compiler_params=pltpu.CompilerParams(
            use_tc_tiling_on_sc=False, needs_layout_passes=False),
        scratch_types=[
            pltpu.VMEM((STATE, _SB), jnp.float32),
            pltpu.VMEM((_SB, STATE), jnp.float32),
        ],
    )
    def tr_kernel(xt_hbm, xlin_hbm, xt_v, xw_v):
        c = lax.axis_index("c")
        s = lax.axis_index("s")
        wid = s * _NC + c
        start = jnp.minimum(wid * per_w - (5 * wid) % 8, n - _SB)
        base = pl.multiple_of(start, 8)
        iota = lax.iota(jnp.int32, 16)
        rowi = iota & 7
        colb = iota >> 3

        for f in range(STATE):
            pltpu.sync_copy(xt_hbm.at[pl.ds(f * n + base, _SB)], xt_v.at[f])

        def tbody(m, carry):
            col = colb + 2 * m
            v = plsc.load_gather(xt_v, [rowi, col])
            plsc.store_scatter(xw_v, [col, rowi], v)
            return carry

        lax.fori_loop(0, _SB * STATE // 16, tbody, 0, unroll=8)
        pltpu.sync_copy(xw_v, xlin_hbm.at[pl.ds(base, _SB)])

    return tr_kernel(xt)


def _sc_segment_sum(x, ei_r, zeros, k_chunks):
    n = x.shape[0]
    zr = n // _NS
    u_n = _UNROLL
    outer = k_chunks // u_n

    @functools.partial(
        pl.kernel,
        out_type=jax.ShapeDtypeStruct((2 * n, STATE), jnp.float32),
        mesh=plsc.VectorSubcoreMesh(core_axis_name="c", subcore_axis_name="s"),
        compiler_params=pltpu.CompilerParams(use_tc_tiling_on_sc=False),
        scratch_types=[
            pltpu.VMEM((k_chunks, _CHUNK), jnp.int32),
            pltpu.VMEM((k_chunks, _CHUNK), jnp.int32),
            pltpu.VMEM((u_n, _CHUNK, STATE), jnp.float32),
            pltpu.VMEM_SHARED((n, STATE), jnp.float32),
            pltpu.SemaphoreType.DMA,
            pltpu.SemaphoreType.DMA,
        ],
    )
    def sc_kernel(x_hbm, ei_hbm, zeros_hbm, p_hbm,
                  src_v, dst_v, rows_v, agg_sh, gsem, ssem):
        c = lax.axis_index("c")
        s = lax.axis_index("s")
        wid = s * _NC + c

        pltpu.sync_copy(zeros_hbm.at[pl.ds(s * zr, zr)],
                        agg_sh.at[pl.ds(s * zr, zr)])
        plsc.subcore_barrier()

        pltpu.sync_copy(ei_hbm.at[0, wid], src_v)
        pltpu.sync_copy(ei_hbm.at[1, wid], dst_v)

        def scatter_descr(u, j):
            return pltpu.make_async_copy(
                rows_v.at[u], agg_sh.at[dst_v.at[j]], ssem)

        def body(jo, carry):
            base = jo * u_n

            @pl.when(jo > 0)
            def _():
                for u in range(u_n):
                    scatter_descr(u, base + u).wait()

            gets = [
                pltpu.async_copy(x_hbm.at[src_v.at[base + u]],
                                 rows_v.at[u], gsem)
                for u in range(u_n)
            ]
            for u in range(u_n):
                gets[u].wait()
                scatter_descr(u, base + u).start(add=True)
            return carry

        lax.fori_loop(0, outer, body, 0)
        for u in range(u_n):
            scatter_descr(u, u).wait()
        plsc.subcore_barrier()

        pltpu.sync_copy(agg_sh.at[pl.ds(s * zr, zr)],
                        p_hbm.at[pl.ds(c * n + s * zr, zr)])

    return sc_kernel(x, ei_r, zeros)


def _mlp_block(x_ref, p0_ref, p1_ref, w1a_ref, w1b_ref, b1_ref, w2_ref,
               b2_ref, o_ref):
    agg = (p0_ref[0] + p1_ref[0]) * (1.0 / 6.0)
    pre = (jnp.dot(x_ref[...], w1a_ref[...], preferred_element_type=jnp.float32)
           + jnp.dot(agg, w1b_ref[...], preferred_element_type=jnp.float32)
           + b1_ref[...])
    h = jnp.tanh(pre)
    o_ref[...] = jnp.tanh(
        jnp.dot(h, w2_ref[...], preferred_element_type=jnp.float32)
        + b2_ref[...])


def _block_diag(w, copies):
    a, b = w.shape
    eye = jnp.eye(copies, dtype=w.dtype)
    return (eye[:, None, :, None] * w[None, :, None, :]).reshape(
        copies * a, copies * b)


def _mlp_wide(xw, pw, w1, b1, w2, b2):
    nw = xw.shape[0]
    hid = w1.shape[1]
    cells = 128 // STATE
    bw = nw
    grid = (nw // bw,)
    p3 = pw.reshape(2, nw, 128)

    w1a = _block_diag(w1[:STATE], cells)
    w1b = _block_diag(w1[STATE:], cells)
    w2d = _block_diag(w2, cells)
    b1t = jnp.tile(b1, cells).reshape(1, cells * hid)
    b2t = jnp.tile(b2, cells).reshape(1, cells * STATE)

    const = lambda shape: pl.BlockSpec(shape, lambda i: (0, 0))
    return pl.pallas_call(
        _mlp_block,
        grid=grid,
        in_specs=[
            pl.BlockSpec((bw, 128), lambda i: (i, 0)),
            pl.BlockSpec((1, bw, 128), lambda i: (0, i, 0)),
            pl.BlockSpec((1, bw, 128), lambda i: (1, i, 0)),
            const((128, cells * hid)), const((128, cells * hid)),
            const((1, cells * hid)),
            const((cells * hid, 128)), const((1, 128)),
        ],
        out_specs=pl.BlockSpec((bw, 128), lambda i: (i, 0)),
        out_shape=jax.ShapeDtypeStruct((nw, 128), jnp.float32),
    )(xw, p3, p3, w1a, w1b, b1t, w2d, b2t)


def kernel(x, edge_index, W1, b1, W2, b2):
    n = x.shape[0]
    e = edge_index.shape[1]
    assert e % (_NW * _CHUNK * _UNROLL) == 0
    assert n % (128 // STATE) == 0 and n % _NS == 0
    assert n % _NW == 0 and _SB >= n // _NW + 8
    k_chunks = e // (_NW * _CHUNK)

    ei_r = edge_index.reshape(2, _NW, k_chunks, _CHUNK)
    zeros = jnp.zeros((n, STATE), jnp.float32)

    x_lin = _sc_transpose(x.T.reshape(-1), n)
    partials = _sc_segment_sum(x_lin, ei_r, zeros, k_chunks)
    nw = n * STATE // 128
    xw = x_lin.reshape(nw, 128)
    pw = partials.reshape(2 * nw, 128)
    out_w = _mlp_wide(xw, pw, W1, b1, W2, b2)
    return out_w.reshape(n, STATE)


# speedup vs baseline: 33.4619x; 1.3255x over previous
import functools

import jax
import jax.numpy as jnp
from jax import lax
from jax.experimental import pallas as pl
from jax.experimental.pallas import tpu as pltpu
from jax.experimental.pallas import tpu_sc as plsc

STATE = 8

_NC = 2
_NS = 16
_NW = _NC * _NS

_CHUNK = 125
_UNROLL = 5
_SB = 3136


def _sc_transpose(xt, n):
    per_w = n // _NW

    @functools.partial(
        pl.kernel,
        out_type=jax.ShapeDtypeStruct((n, STATE), jnp.float32),
        mesh=plsc.VectorSubcoreMesh(core_axis_name="c", subcore_axis_name="s"),
        compiler_params=pltpu.CompilerParams(
            use_tc_tiling_on_sc=False, needs_layout_passes=False),
        scratch_types=[
            pltpu.VMEM((STATE, _SB), jnp.float32),
            pltpu.VMEM((_SB, STATE), jnp.float32),
        ],
    )
    def tr_kernel(xt_hbm, xlin_hbm, xt_v, xw_v):
        c = lax.axis_index("c")
        s = lax.axis_index("s")
        wid = s * _NC + c
        start = jnp.minimum(wid * per_w - (5 * wid) % 8, n - _SB)
        base = pl.multiple_of(start, 8)
        iota = lax.iota(jnp.int32, 16)
        rowi = iota & 7
        colb = iota >> 3

        for f in range(STATE):
            pltpu.sync_copy(xt_hbm.at[pl.ds(f * n + base, _SB)], xt_v.at[f])

        def tbody(m, carry):
            col = colb + 2 * m
            v = plsc.load_gather(xt_v, [rowi, col])
            plsc.store_scatter(xw_v, [col, rowi], v)
            return carry

        lax.fori_loop(0, _SB * STATE // 16, tbody, 0, unroll=8)
        pltpu.sync_copy(xw_v, xlin_hbm.at[pl.ds(base, _SB)])

    return tr_kernel(xt)


def _sc_transpose_out(ow, n):
    per_w = n // _NW

    @functools.partial(
        pl.kernel,
        out_type=jax.ShapeDtypeStruct((STATE * n,), jnp.float32),
        mesh=plsc.VectorSubcoreMesh(core_axis_name="c", subcore_axis_name="s"),
        compiler_params=pltpu.CompilerParams(
            use_tc_tiling_on_sc=False, needs_layout_passes=False),
        scratch_types=[
            pltpu.VMEM((_SB, STATE), jnp.float32),
            pltpu.VMEM((STATE, _SB), jnp.float32),
        ],
    )
    def tr_kernel(ow_hbm, of_hbm, xw_v, xt_v):
        c = lax.axis_index("c")
        s = lax.axis_index("s")
        wid = s * _NC + c
        start = jnp.minimum(wid * per_w - (5 * wid) % 8, n - _SB)
        base = pl.multiple_of(start, 8)
        iota = lax.iota(jnp.int32, 16)
        rowi = iota & 7
        colb = iota >> 3

        pltpu.sync_copy(ow_hbm.at[pl.ds(base, _SB)], xw_v)

        def tbody(m, carry):
            col = colb + 2 * m
            v = plsc.load_gather(xw_v, [col, rowi])
            plsc.store_scatter(xt_v, [rowi, col], v)
            return carry

        lax.fori_loop(0, _SB * STATE // 16, tbody, 0, unroll=8)
        for f in range(STATE):
            pltpu.sync_copy(xt_v.at[f], of_hbm.at[pl.ds(f * n + base, _SB)])

    return tr_kernel(ow)


def _sc_segment_sum(x, ei_r, zeros, k_chunks):
    n = x.shape[0]
    zr = n // _NS
    u_n = _UNROLL
    outer = k_chunks // u_n

    @functools.partial(
        pl.kernel,
        out_type=jax.ShapeDtypeStruct((2 * n, STATE), jnp.float32),
        mesh=plsc.VectorSubcoreMesh(core_axis_name="c", subcore_axis_name="s"),
        compiler_params=pltpu.CompilerParams(use_tc_tiling_on_sc=False),
        scratch_types=[
            pltpu.VMEM((k_chunks, _CHUNK), jnp.int32),
            pltpu.VMEM((k_chunks, _CHUNK), jnp.int32),
            pltpu.VMEM((u_n, _CHUNK, STATE), jnp.float32),
            pltpu.VMEM_SHARED((n, STATE), jnp.float32),
            pltpu.SemaphoreType.DMA,
            pltpu.SemaphoreType.DMA,
        ],
    )
    def sc_kernel(x_hbm, ei_hbm, zeros_hbm, p_hbm,
                  src_v, dst_v, rows_v, agg_sh, gsem, ssem):
        c = lax.axis_index("c")
        s = lax.axis_index("s")
        wid = s * _NC + c

        pltpu.sync_copy(zeros_hbm.at[pl.ds(s * zr, zr)],
                        agg_sh.at[pl.ds(s * zr, zr)])
        plsc.subcore_barrier()

        pltpu.sync_copy(ei_hbm.at[0, wid], src_v)
        pltpu.sync_copy(ei_hbm.at[1, wid], dst_v)

        def scatter_descr(u, j):
            return pltpu.make_async_copy(
                rows_v.at[u], agg_sh.at[dst_v.at[j]], ssem)

        def body(jo, carry):
            base = jo * u_n

            @pl.when(jo > 0)
            def _():
                for u in range(u_n):
                    scatter_descr(u, base + u).wait()

            gets = [
                pltpu.async_copy(x_hbm.at[src_v.at[base + u]],
                                 rows_v.at[u], gsem)
                for u in range(u_n)
            ]
            for u in range(u_n):
                gets[u].wait()
                scatter_descr(u, base + u).start(add=True)
            return carry

        lax.fori_loop(0, outer, body, 0)
        for u in range(u_n):
            scatter_descr(u, u).wait()
        plsc.subcore_barrier()

        pltpu.sync_copy(agg_sh.at[pl.ds(s * zr, zr)],
                        p_hbm.at[pl.ds(c * n + s * zr, zr)])

    return sc_kernel(x, ei_r, zeros)


def _mlp_block(x_ref, p0_ref, p1_ref, w1a_ref, w1b_ref, b1_ref, w2_ref,
               b2_ref, o_ref):
    agg = (p0_ref[0] + p1_ref[0]) * (1.0 / 6.0)
    pre = (jnp.dot(x_ref[...], w1a_ref[...], preferred_element_type=jnp.float32)
           + jnp.dot(agg, w1b_ref[...], preferred_element_type=jnp.float32)
           + b1_ref[...])
    h = jnp.tanh(pre)
    o_ref[...] = jnp.tanh(
        jnp.dot(h, w2_ref[...], preferred_element_type=jnp.float32)
        + b2_ref[...])


def _block_diag(w, copies):
    a, b = w.shape
    eye = jnp.eye(copies, dtype=w.dtype)
    return (eye[:, None, :, None] * w[None, :, None, :]).reshape(
        copies * a, copies * b)


def _mlp_wide(xw, pw, w1, b1, w2, b2):
    nw = xw.shape[0]
    hid = w1.shape[1]
    cells = 128 // STATE
    bw = nw
    grid = (nw // bw,)
    p3 = pw.reshape(2, nw, 128)

    w1a = _block_diag(w1[:STATE], cells)
    w1b = _block_diag(w1[STATE:], cells)
    w2d = _block_diag(w2, cells)
    b1t = jnp.tile(b1, cells).reshape(1, cells * hid)
    b2t = jnp.tile(b2, cells).reshape(1, cells * STATE)

    const = lambda shape: pl.BlockSpec(shape, lambda i: (0, 0))
    return pl.pallas_call(
        _mlp_block,
        grid=grid,
        in_specs=[
            pl.BlockSpec((bw, 128), lambda i: (i, 0)),
            pl.BlockSpec((1, bw, 128), lambda i: (0, i, 0)),
            pl.BlockSpec((1, bw, 128), lambda i: (1, i, 0)),
            const((128, cells * hid)), const((128, cells * hid)),
            const((1, cells * hid)),
            const((cells * hid, 128)), const((1, 128)),
        ],
        out_specs=pl.BlockSpec((bw, 128), lambda i: (i, 0)),
        out_shape=jax.ShapeDtypeStruct((nw, 128), jnp.float32),
    )(xw, p3, p3, w1a, w1b, b1t, w2d, b2t)


def kernel(x, edge_index, W1, b1, W2, b2):
    n = x.shape[0]
    e = edge_index.shape[1]
    assert e % (_NW * _CHUNK * _UNROLL) == 0
    assert n % (128 // STATE) == 0 and n % _NS == 0
    assert n % _NW == 0 and _SB >= n // _NW + 8
    k_chunks = e // (_NW * _CHUNK)

    ei_r = edge_index.reshape(2, _NW, k_chunks, _CHUNK)
    zeros = jnp.zeros((n, STATE), jnp.float32)

    x_lin = _sc_transpose(x.T.reshape(-1), n)
    partials = _sc_segment_sum(x_lin, ei_r, zeros, k_chunks)
    nw = n * STATE // 128
    xw = x_lin.reshape(nw, 128)
    pw = partials.reshape(2 * nw, 128)
    out_w = _mlp_wide(xw, pw, W1, b1, W2, b2)
    out_f = _sc_transpose_out(out_w.reshape(n, STATE), n)
    return out_f.reshape(STATE, n).T
